# baseline (speedup 1.0000x reference)
import jax
import jax.numpy as jnp
from jax import lax
from jax.experimental import pallas as pl
from jax.experimental.pallas import tpu as pltpu

N = 10000
D_IN = 256
D_HID = 256
D_OUT = 64
BM = 400
HB = BM // 2
NB = N // BM


def _body(x_ref, a0_ref, a1_ref, w1_ref, w2_ref, w3_ref, out_ref, s_a, s_b):
    i = pl.program_id(0)
    t = i - 1
    j = t % NB
    layer = t // NB
    row = j * BM

    @pl.when(i == 0)
    def _():
        def chunk(k, carry):
            r = k * BM
            s_a[pl.ds(r, BM), :] = jnp.dot(
                x_ref[pl.ds(r, BM), :], w1_ref[...],
                preferred_element_type=jnp.float32)
            return carry
        lax.fori_loop(0, NB, chunk, 0)

    @pl.when(layer == 0)
    def _():
        h0 = jnp.maximum(jnp.dot(a0_ref[...], s_a[...],
                                 preferred_element_type=jnp.float32), 0.0)
        s_b[pl.ds(row, HB), :] = jnp.dot(
            h0, w2_ref[...], preferred_element_type=jnp.float32)
        h1 = jnp.maximum(jnp.dot(a1_ref[...], s_a[...],
                                 preferred_element_type=jnp.float32), 0.0)
        s_b[pl.ds(row + HB, HB), :] = jnp.dot(
            h1, w2_ref[...], preferred_element_type=jnp.float32)

    @pl.when(layer == 1)
    def _():
        h0 = jnp.maximum(jnp.dot(a0_ref[...], s_b[...],
                                 preferred_element_type=jnp.float32), 0.0)
        s_a[pl.ds(row, HB), :D_OUT] = jnp.dot(
            h0, w3_ref[...], preferred_element_type=jnp.float32)
        h1 = jnp.maximum(jnp.dot(a1_ref[...], s_b[...],
                                 preferred_element_type=jnp.float32), 0.0)
        s_a[pl.ds(row + HB, HB), :D_OUT] = jnp.dot(
            h1, w3_ref[...], preferred_element_type=jnp.float32)

    @pl.when(layer == 2)
    def _():
        def smax(h):
            m = jnp.max(h, axis=-1, keepdims=True)
            e = jnp.exp(h - m)
            return e / jnp.sum(e, axis=-1, keepdims=True)
        h0 = jnp.maximum(jnp.dot(a0_ref[...], s_a[:, :D_OUT],
                                 preferred_element_type=jnp.float32), 0.0)
        out_ref[:HB, :] = smax(h0)
        h1 = jnp.maximum(jnp.dot(a1_ref[...], s_a[:, :D_OUT],
                                 preferred_element_type=jnp.float32), 0.0)
        out_ref[HB:, :] = smax(h1)


def _band0_idx(i):
    return (2 * (jnp.maximum(i - 1, 0) % NB), 0)


def _band1_idx(i):
    return (2 * (jnp.maximum(i - 1, 0) % NB) + 1, 0)


def _out_idx(i):
    return (jnp.maximum(i - 1 - 2 * NB, 0), 0)


def kernel(input, adj, W1, W2, W3):
    return pl.pallas_call(
        _body,
        grid=(1 + 3 * NB,),
        in_specs=[
            pl.BlockSpec((N, D_IN), lambda i: (0, 0)),
            pl.BlockSpec((HB, N), _band0_idx),
            pl.BlockSpec((HB, N), _band1_idx),
            pl.BlockSpec((D_IN, D_HID), lambda i: (0, 0)),
            pl.BlockSpec((D_HID, D_HID), lambda i: (0, 0)),
            pl.BlockSpec((D_HID, D_OUT), lambda i: (0, 0)),
        ],
        out_specs=pl.BlockSpec((BM, D_OUT), _out_idx),
        out_shape=jax.ShapeDtypeStruct((N, D_OUT), jnp.float32),
        scratch_shapes=[
            pltpu.VMEM((N, D_HID), jnp.float32),
            pltpu.VMEM((N, D_HID), jnp.float32),
        ],
        compiler_params=pltpu.CompilerParams(
            dimension_semantics=("arbitrary",),
            vmem_limit_bytes=128 * 1024 * 1024,
        ),
    )(input, adj, adj, W1, W2, W3)
